# Initial kernel scaffold; baseline (speedup 1.0000x reference)
import jax
import jax.numpy as jnp
from jax.experimental import pallas as pl


def kernel(sample, hidden, cell_tensor, d_0, Wc, bc, Wi, bi, Wf, bf, Wo, bo, Wout, bout):
    raise NotImplementedError("write your pallas kernel here")


# speedup vs baseline: 2.4486x; 2.4486x over previous
import jax
import jax.numpy as jnp
from jax.experimental import pallas as pl
from jax.experimental.pallas import tpu as pltpu


def _gates_kernel(x_ref, h_ref, d0_ref, wxh_ref, wfd_ref, b4_ref,
                  d_ref, ci_ref, og_ref, g4):
    n_in = x_ref.shape[1]
    hdim = d_ref.shape[1]
    g4[...] = (
        jnp.dot(x_ref[...], wxh_ref[0:n_in], preferred_element_type=jnp.float32)
        + jnp.dot(h_ref[...], wxh_ref[n_in:], preferred_element_type=jnp.float32)
        + b4_ref[...]
    )
    f_pre = g4[:, 3 * hdim:] + jnp.dot(
        d0_ref[...], wfd_ref[...], preferred_element_type=jnp.float32)
    d_ref[...] = jax.nn.sigmoid(f_pre) * 0.5
    ci_ref[...] = jnp.tanh(g4[:, 2 * hdim:3 * hdim]) * jax.nn.sigmoid(g4[:, 0:hdim])
    og_ref[...] = jax.nn.sigmoid(g4[:, hdim:2 * hdim])


def _mem_kernel(cell_ref, d_ref, ci_ref, og_ref, wout_ref, bout_ref,
                hc_ref, hn_ref, out_ref):
    k = cell_ref.shape[0]
    d = d_ref[...]
    w = None
    acc = None
    for n in range(k):
        j = k - 1 - n
        c = cell_ref[j]
        if j >= 1:
            hc_ref[j - 1] = c
        if n == 0:
            w = -d
            acc = c * w
        else:
            w = w * ((float(n) - d) * (1.0 / (n + 1.0)))
            acc = acc + c * w
    cell = ci_ref[...] - acc
    hc_ref[k - 1] = cell
    hn = jnp.tanh(cell) * og_ref[...]
    hn_ref[...] = hn
    out_ref[...] = jnp.dot(
        hn, wout_ref[...], preferred_element_type=jnp.float32) + bout_ref[...]


def kernel(sample, hidden, cell_tensor, d_0, Wc, bc, Wi, bi, Wf, bf, Wo, bo,
           Wout, bout, *, interpret=False):
    k, b, h = cell_tensor.shape
    n_in = sample.shape[1]
    out_dim = Wout.shape[0]

    wxh = jnp.concatenate(
        [Wi.T, Wo.T, Wc.T, Wf[:, :n_in + h].T], axis=1)
    wfd = Wf[:, n_in + h:].T
    b4 = jnp.concatenate([bi, bo, bc, bf]).reshape(1, 4 * h)
    woutT = Wout.T
    bout2 = bout.reshape(1, out_dim)

    bg = 256
    d_values, ci, og = pl.pallas_call(
        _gates_kernel,
        grid=(b // bg,),
        in_specs=[
            pl.BlockSpec((bg, n_in), lambda i: (i, 0)),
            pl.BlockSpec((bg, h), lambda i: (i, 0)),
            pl.BlockSpec((bg, h), lambda i: (i, 0)),
            pl.BlockSpec((n_in + h, 4 * h), lambda i: (0, 0)),
            pl.BlockSpec((h, h), lambda i: (0, 0)),
            pl.BlockSpec((1, 4 * h), lambda i: (0, 0)),
        ],
        out_specs=[
            pl.BlockSpec((bg, h), lambda i: (i, 0)),
            pl.BlockSpec((bg, h), lambda i: (i, 0)),
            pl.BlockSpec((bg, h), lambda i: (i, 0)),
        ],
        out_shape=[jax.ShapeDtypeStruct((b, h), jnp.float32)] * 3,
        scratch_shapes=[pltpu.VMEM((bg, 4 * h), jnp.float32)],
        compiler_params=pltpu.CompilerParams(
            dimension_semantics=("parallel",),
            vmem_limit_bytes=50 * 1024 * 1024,
        ),
        name="mlstm_gates",
        interpret=interpret,
    )(sample, hidden, d_0, wxh, wfd, b4)

    bm = 128
    hc, hidden_new, output = pl.pallas_call(
        _mem_kernel,
        grid=(b // bm,),
        in_specs=[
            pl.BlockSpec((k, bm, h), lambda i: (0, i, 0)),
            pl.BlockSpec((bm, h), lambda i: (i, 0)),
            pl.BlockSpec((bm, h), lambda i: (i, 0)),
            pl.BlockSpec((bm, h), lambda i: (i, 0)),
            pl.BlockSpec((h, out_dim), lambda i: (0, 0)),
            pl.BlockSpec((1, out_dim), lambda i: (0, 0)),
        ],
        out_specs=[
            pl.BlockSpec((k, bm, h), lambda i: (0, i, 0)),
            pl.BlockSpec((bm, h), lambda i: (i, 0)),
            pl.BlockSpec((bm, out_dim), lambda i: (i, 0)),
        ],
        out_shape=[
            jax.ShapeDtypeStruct((k, b, h), jnp.float32),
            jax.ShapeDtypeStruct((b, h), jnp.float32),
            jax.ShapeDtypeStruct((b, out_dim), jnp.float32),
        ],
        compiler_params=pltpu.CompilerParams(
            dimension_semantics=("parallel",),
            vmem_limit_bytes=50 * 1024 * 1024,
        ),
        name="mlstm_mem",
        interpret=interpret,
    )(cell_tensor, d_values, ci, og, woutT, bout2)

    return (output, hidden_new, hc, d_values)


# speedup vs baseline: 2.5854x; 1.0559x over previous
import jax
import jax.numpy as jnp
from jax.experimental import pallas as pl
from jax.experimental.pallas import tpu as pltpu


def _mlstm_kernel(x_ref, h_ref, d0_ref, cell_ref, wxh_ref, wfd_ref, b4_ref,
                  wout_ref, bout_ref,
                  out_ref, hn_ref, hc_ref, d_ref, g4):
    n_in = x_ref.shape[1]
    hdim = h_ref.shape[1]
    k = cell_ref.shape[0]

    g4[...] = (
        jnp.dot(x_ref[...], wxh_ref[0:n_in], preferred_element_type=jnp.float32)
        + jnp.dot(h_ref[...], wxh_ref[n_in:], preferred_element_type=jnp.float32)
        + b4_ref[...]
    )
    f_pre = g4[:, 3 * hdim:] + jnp.dot(
        d0_ref[...], wfd_ref[...], preferred_element_type=jnp.float32)
    d = jax.nn.sigmoid(f_pre) * 0.5
    d_ref[...] = d
    ci = jnp.tanh(g4[:, 2 * hdim:3 * hdim]) * jax.nn.sigmoid(g4[:, 0:hdim])

    w = None
    acc = None
    for n in range(k):
        j = k - 1 - n
        c = cell_ref[j]
        if j >= 1:
            hc_ref[j - 1] = c
        if n == 0:
            w = -d
            acc = c * w
        else:
            w = w * ((float(n) - d) * (1.0 / (n + 1.0)))
            acc = acc + c * w
    cell = ci - acc
    hc_ref[k - 1] = cell
    hn = jnp.tanh(cell) * jax.nn.sigmoid(g4[:, hdim:2 * hdim])
    hn_ref[...] = hn
    out_ref[...] = jnp.dot(
        hn, wout_ref[...], preferred_element_type=jnp.float32) + bout_ref[...]


def kernel(sample, hidden, cell_tensor, d_0, Wc, bc, Wi, bi, Wf, bf, Wo, bo,
           Wout, bout, *, interpret=False):
    k, b, h = cell_tensor.shape
    n_in = sample.shape[1]
    out_dim = Wout.shape[0]

    wxh = jnp.concatenate(
        [Wi.T, Wo.T, Wc.T, Wf[:, :n_in + h].T], axis=1)
    wfd = Wf[:, n_in + h:].T
    b4 = jnp.concatenate([bi, bo, bc, bf]).reshape(1, 4 * h)
    woutT = Wout.T
    bout2 = bout.reshape(1, out_dim)

    bm = 128
    output, hidden_new, hc, d_values = pl.pallas_call(
        _mlstm_kernel,
        grid=(b // bm,),
        in_specs=[
            pl.BlockSpec((bm, n_in), lambda i: (i, 0)),
            pl.BlockSpec((bm, h), lambda i: (i, 0)),
            pl.BlockSpec((bm, h), lambda i: (i, 0)),
            pl.BlockSpec((k, bm, h), lambda i: (0, i, 0)),
            pl.BlockSpec((n_in + h, 4 * h), lambda i: (0, 0)),
            pl.BlockSpec((h, h), lambda i: (0, 0)),
            pl.BlockSpec((1, 4 * h), lambda i: (0, 0)),
            pl.BlockSpec((h, out_dim), lambda i: (0, 0)),
            pl.BlockSpec((1, out_dim), lambda i: (0, 0)),
        ],
        out_specs=[
            pl.BlockSpec((bm, out_dim), lambda i: (i, 0)),
            pl.BlockSpec((bm, h), lambda i: (i, 0)),
            pl.BlockSpec((k, bm, h), lambda i: (0, i, 0)),
            pl.BlockSpec((bm, h), lambda i: (i, 0)),
        ],
        out_shape=[
            jax.ShapeDtypeStruct((b, out_dim), jnp.float32),
            jax.ShapeDtypeStruct((b, h), jnp.float32),
            jax.ShapeDtypeStruct((k, b, h), jnp.float32),
            jax.ShapeDtypeStruct((b, h), jnp.float32),
        ],
        scratch_shapes=[pltpu.VMEM((bm, 4 * h), jnp.float32)],
        compiler_params=pltpu.CompilerParams(
            dimension_semantics=("parallel",),
            vmem_limit_bytes=58 * 1024 * 1024,
        ),
        name="mlstm_fused",
        interpret=interpret,
    )(sample, hidden, d_0, cell_tensor, wxh, wfd, b4, woutT, bout2)

    return (output, hidden_new, hc, d_values)


# speedup vs baseline: 2.7849x; 1.0772x over previous
import jax
import jax.numpy as jnp
from jax.experimental import pallas as pl
from jax.experimental.pallas import tpu as pltpu


def _mlstm_kernel(x_ref, h_ref, d0_ref, cell_ref, wxh_ref, wfd_ref, b4_ref,
                  wout_ref, bout_ref,
                  out_ref, hn_ref, hc_ref, d_ref, g4):
    n_in = x_ref.shape[1]
    hdim = h_ref.shape[1]
    k = cell_ref.shape[0]

    bf16 = jnp.bfloat16
    g4[...] = (
        jnp.dot(x_ref[...].astype(bf16), wxh_ref[0:n_in],
                preferred_element_type=jnp.float32)
        + jnp.dot(h_ref[...].astype(bf16), wxh_ref[n_in:],
                  preferred_element_type=jnp.float32)
        + b4_ref[...]
    )
    f_pre = g4[:, 3 * hdim:] + jnp.dot(
        d0_ref[...].astype(bf16), wfd_ref[...],
        preferred_element_type=jnp.float32)
    d = jax.nn.sigmoid(f_pre) * 0.5
    d_ref[...] = d
    ci = jnp.tanh(g4[:, 2 * hdim:3 * hdim]) * jax.nn.sigmoid(g4[:, 0:hdim])

    w = None
    acc = None
    for n in range(k):
        j = k - 1 - n
        c = cell_ref[j]
        if j >= 1:
            hc_ref[j - 1] = c
        if n == 0:
            w = -d
            acc = c * w
        else:
            w = w * ((float(n) - d) * (1.0 / (n + 1.0)))
            acc = acc + c * w
    cell = ci - acc
    hc_ref[k - 1] = cell
    hn = jnp.tanh(cell) * jax.nn.sigmoid(g4[:, hdim:2 * hdim])
    hn_ref[...] = hn
    out_ref[...] = jnp.dot(
        hn.astype(jnp.bfloat16), wout_ref[...],
        preferred_element_type=jnp.float32) + bout_ref[...]


def kernel(sample, hidden, cell_tensor, d_0, Wc, bc, Wi, bi, Wf, bf, Wo, bo,
           Wout, bout, *, interpret=False):
    k, b, h = cell_tensor.shape
    n_in = sample.shape[1]
    out_dim = Wout.shape[0]

    wxh = jnp.concatenate(
        [Wi.T, Wo.T, Wc.T, Wf[:, :n_in + h].T],
        axis=1).astype(jnp.bfloat16)
    wfd = Wf[:, n_in + h:].T.astype(jnp.bfloat16)
    b4 = jnp.concatenate([bi, bo, bc, bf]).reshape(1, 4 * h)
    woutT = Wout.T.astype(jnp.bfloat16)
    bout2 = bout.reshape(1, out_dim)

    bm = 128
    output, hidden_new, hc, d_values = pl.pallas_call(
        _mlstm_kernel,
        grid=(b // bm,),
        in_specs=[
            pl.BlockSpec((bm, n_in), lambda i: (i, 0)),
            pl.BlockSpec((bm, h), lambda i: (i, 0)),
            pl.BlockSpec((bm, h), lambda i: (i, 0)),
            pl.BlockSpec((k, bm, h), lambda i: (0, i, 0)),
            pl.BlockSpec((n_in + h, 4 * h), lambda i: (0, 0)),
            pl.BlockSpec((h, h), lambda i: (0, 0)),
            pl.BlockSpec((1, 4 * h), lambda i: (0, 0)),
            pl.BlockSpec((h, out_dim), lambda i: (0, 0)),
            pl.BlockSpec((1, out_dim), lambda i: (0, 0)),
        ],
        out_specs=[
            pl.BlockSpec((bm, out_dim), lambda i: (i, 0)),
            pl.BlockSpec((bm, h), lambda i: (i, 0)),
            pl.BlockSpec((k, bm, h), lambda i: (0, i, 0)),
            pl.BlockSpec((bm, h), lambda i: (i, 0)),
        ],
        out_shape=[
            jax.ShapeDtypeStruct((b, out_dim), jnp.float32),
            jax.ShapeDtypeStruct((b, h), jnp.float32),
            jax.ShapeDtypeStruct((k, b, h), jnp.float32),
            jax.ShapeDtypeStruct((b, h), jnp.float32),
        ],
        scratch_shapes=[pltpu.VMEM((bm, 4 * h), jnp.float32)],
        compiler_params=pltpu.CompilerParams(
            dimension_semantics=("parallel",),
            vmem_limit_bytes=58 * 1024 * 1024,
        ),
        name="mlstm_fused",
        interpret=interpret,
    )(sample, hidden, d_0, cell_tensor, wxh, wfd, b4, woutT, bout2)

    return (output, hidden_new, hc, d_values)


# speedup vs baseline: 2.8160x; 1.0112x over previous
import jax
import jax.numpy as jnp
from jax.experimental import pallas as pl
from jax.experimental.pallas import tpu as pltpu


def _dot_nk(a, w_nk):
    return jax.lax.dot_general(
        a, w_nk, (((1,), (1,)), ((), ())),
        preferred_element_type=jnp.float32)


def _mlstm_kernel(x_ref, h_ref, d0_ref, cell_ref, wxh_ref, wfd_ref, b4_ref,
                  wout_ref, bout_ref,
                  out_ref, hn_ref, hc_ref, d_ref, g4):
    n_in = x_ref.shape[1]
    hdim = h_ref.shape[1]
    k = cell_ref.shape[0]
    bf16 = jnp.bfloat16

    g4[...] = (
        _dot_nk(x_ref[...].astype(bf16), wxh_ref[:, 0:n_in])
        + _dot_nk(h_ref[...].astype(bf16), wxh_ref[:, n_in:])
        + b4_ref[...]
    )
    f_pre = g4[:, 3 * hdim:] + _dot_nk(d0_ref[...].astype(bf16), wfd_ref[...])
    d = jax.nn.sigmoid(f_pre) * 0.5
    d_ref[...] = d
    ci = jnp.tanh(g4[:, 2 * hdim:3 * hdim]) * jax.nn.sigmoid(g4[:, 0:hdim])

    w = None
    acc = None
    for n in range(k):
        j = k - 1 - n
        c = cell_ref[j]
        if j >= 1:
            hc_ref[j - 1] = c
        if n == 0:
            w = -d
            acc = c * w
        else:
            w = w * ((float(n) - d) * (1.0 / (n + 1.0)))
            acc = acc + c * w
    cell = ci - acc
    hc_ref[k - 1] = cell
    hn = jnp.tanh(cell) * jax.nn.sigmoid(g4[:, hdim:2 * hdim])
    hn_ref[...] = hn
    out_ref[...] = _dot_nk(hn.astype(bf16), wout_ref[...]) + bout_ref[...]


def kernel(sample, hidden, cell_tensor, d_0, Wc, bc, Wi, bi, Wf, bf, Wo, bo,
           Wout, bout, *, interpret=False):
    k, b, h = cell_tensor.shape
    n_in = sample.shape[1]
    out_dim = Wout.shape[0]

    wxh = jnp.concatenate(
        [Wi, Wo, Wc, Wf[:, :n_in + h]], axis=0).astype(jnp.bfloat16)
    wfd = Wf[:, n_in + h:].astype(jnp.bfloat16)
    b4 = jnp.concatenate([bi, bo, bc, bf]).reshape(1, 4 * h)
    wout_nk = Wout.astype(jnp.bfloat16)
    bout2 = bout.reshape(1, out_dim)

    bm = 128
    output, hidden_new, hc, d_values = pl.pallas_call(
        _mlstm_kernel,
        grid=(b // bm,),
        in_specs=[
            pl.BlockSpec((bm, n_in), lambda i: (i, 0)),
            pl.BlockSpec((bm, h), lambda i: (i, 0)),
            pl.BlockSpec((bm, h), lambda i: (i, 0)),
            pl.BlockSpec((k, bm, h), lambda i: (0, i, 0)),
            pl.BlockSpec((4 * h, n_in + h), lambda i: (0, 0)),
            pl.BlockSpec((h, h), lambda i: (0, 0)),
            pl.BlockSpec((1, 4 * h), lambda i: (0, 0)),
            pl.BlockSpec((out_dim, h), lambda i: (0, 0)),
            pl.BlockSpec((1, out_dim), lambda i: (0, 0)),
        ],
        out_specs=[
            pl.BlockSpec((bm, out_dim), lambda i: (i, 0)),
            pl.BlockSpec((bm, h), lambda i: (i, 0)),
            pl.BlockSpec((k, bm, h), lambda i: (0, i, 0)),
            pl.BlockSpec((bm, h), lambda i: (i, 0)),
        ],
        out_shape=[
            jax.ShapeDtypeStruct((b, out_dim), jnp.float32),
            jax.ShapeDtypeStruct((b, h), jnp.float32),
            jax.ShapeDtypeStruct((k, b, h), jnp.float32),
            jax.ShapeDtypeStruct((b, h), jnp.float32),
        ],
        scratch_shapes=[pltpu.VMEM((bm, 4 * h), jnp.float32)],
        compiler_params=pltpu.CompilerParams(
            dimension_semantics=("parallel",),
            vmem_limit_bytes=58 * 1024 * 1024,
        ),
        name="mlstm_fused",
        interpret=interpret,
    )(sample, hidden, d_0, cell_tensor, wxh, wfd, b4, wout_nk, bout2)

    return (output, hidden_new, hc, d_values)
